# baseline (speedup 1.0000x reference)
import functools

import jax
import jax.numpy as jnp
from jax import lax
from jax.experimental import pallas as pl
from jax.experimental.pallas import tpu as pltpu
from jax.experimental.pallas import tpu_sc as plsc

B = 4096
C = 30
CP = 32
D = 64
NT = 113

NC = 2
NS = 16
NW = NC * NS
BPW = B // NW
CHUNK = 16
NCHUNK = BPW // CHUNK
IDXC = CHUNK * C
GSL = 120
NG = IDXC // GSL


def _sc_body(table_hbm, widx_hbm, ctx_hbm, out_hbm,
             idx_v, rows_v, ctx_v, pscr_v, sco_v, sem):
    wid = lax.axis_index("s") * NC + lax.axis_index("c")
    zero16 = jnp.zeros((16,), jnp.float32)
    pscr_v[pl.ds(C * 16, 16)] = zero16
    pscr_v[pl.ds((C + 1) * 16, 16)] = zero16
    iota16 = lax.iota(jnp.int32, 16)

    for k in range(NCHUNK):
        bbase = wid * BPW + k * CHUNK
        ioff = pl.multiple_of(bbase * C, 8)
        pltpu.sync_copy(widx_hbm.at[pl.ds(ioff, IDXC)], idx_v)
        pltpu.sync_copy(ctx_hbm.at[pl.ds(bbase, CHUNK)], ctx_v)
        cps = [
            pltpu.async_copy(
                table_hbm.at[idx_v.at[pl.ds(j * GSL, GSL)]],
                rows_v.at[pl.ds(j * GSL, GSL)], sem)
            for j in range(NG)
        ]
        for cp in cps:
            cp.wait()

        def body_b(b, _):
            ctx = [ctx_v[b, pl.ds(i * 16, 16)] for i in range(4)]
            r0 = b * C
            for c in range(C):
                acc = rows_v[r0 + c, pl.ds(0, 16)] * ctx[0]
                for i in range(1, 4):
                    acc = acc + rows_v[r0 + c, pl.ds(i * 16, 16)] * ctx[i]
                pscr_v[pl.ds(c * 16, 16)] = acc
            for g in range(2):
                base_idx = iota16 * 16
                acc = jnp.zeros((16,), jnp.float32)
                for l in range(16):
                    acc = acc + plsc.load_gather(
                        pscr_v, [base_idx + (g * 256 + l)])
                sco_v[b, pl.ds(g * 16, 16)] = acc
            return 0

        lax.fori_loop(0, CHUNK, body_b, 0)
        pltpu.sync_copy(sco_v, out_hbm.at[pl.ds(bbase, CHUNK)])


def _sc_scores(table, widx_flat, ctx):
    mesh = plsc.VectorSubcoreMesh(core_axis_name="c", subcore_axis_name="s")
    fn = pl.kernel(
        _sc_body,
        out_type=jax.ShapeDtypeStruct((B, CP), jnp.float32),
        mesh=mesh,
        scratch_types=[
            pltpu.VMEM((IDXC,), jnp.int32),
            pltpu.VMEM((IDXC, D), jnp.float32),
            pltpu.VMEM((CHUNK, D), jnp.float32),
            pltpu.VMEM((CP * 16,), jnp.float32),
            pltpu.VMEM((CHUNK, CP), jnp.float32),
            pltpu.SemaphoreType.DMA,
        ],
        compiler_params=pltpu.CompilerParams(
            needs_layout_passes=False, use_tc_tiling_on_sc=False),
    )
    return fn(table, widx_flat, ctx)


def _tc_body(ctx_ref, w_ref, b_ref, sco_ref, scores_ref, probs_ref, mt_ref):
    z = jnp.dot(ctx_ref[...], w_ref[...], preferred_element_type=jnp.float32)
    z = z + b_ref[...]
    mt_ref[...] = jax.nn.sigmoid(z)
    s = sco_ref[...][:, :C]
    m = jnp.max(s, axis=1, keepdims=True)
    e = jnp.exp(s - m)
    probs_ref[...] = e / jnp.sum(e, axis=1, keepdims=True)
    scores_ref[...] = s


def _tc_finish(ctx, type_W, type_b, sco_pad):
    nblk = 8
    blk = B // nblk
    return pl.pallas_call(
        _tc_body,
        grid=(nblk,),
        in_specs=[
            pl.BlockSpec((blk, D), lambda i: (i, 0)),
            pl.BlockSpec((D, NT), lambda i: (0, 0)),
            pl.BlockSpec((1, NT), lambda i: (0, 0)),
            pl.BlockSpec((blk, CP), lambda i: (i, 0)),
        ],
        out_specs=[
            pl.BlockSpec((blk, C), lambda i: (i, 0)),
            pl.BlockSpec((blk, C), lambda i: (i, 0)),
            pl.BlockSpec((blk, NT), lambda i: (i, 0)),
        ],
        out_shape=[
            jax.ShapeDtypeStruct((B, C), jnp.float32),
            jax.ShapeDtypeStruct((B, C), jnp.float32),
            jax.ShapeDtypeStruct((B, NT), jnp.float32),
        ],
    )(ctx, type_W, type_b.reshape(1, NT), sco_pad)


def kernel(leftb, rightb, leftlens, rightlens, docb, wididxsb,
           entity_table, context_encoded, type_W, type_b):
    widx_flat = wididxsb.reshape(-1)
    sco_pad = _sc_scores(entity_table, widx_flat, context_encoded)
    scores, probs, mtype = _tc_finish(context_encoded, type_W, type_b, sco_pad)
    return scores, probs, mtype
